# Initial kernel scaffold; baseline (speedup 1.0000x reference)
import jax
import jax.numpy as jnp
from jax.experimental import pallas as pl


def kernel(lncrna_x, disease_x, adj, W, attn_l, attn_r, bias):
    raise NotImplementedError("write your pallas kernel here")


# speedup vs baseline: 1.6945x; 1.6945x over previous
import functools

import jax
import jax.numpy as jnp
from jax import lax
from jax.experimental import pallas as pl
from jax.experimental.pallas import tpu as pltpu

LNC = 5000
DIS = 3000
N = LNC + DIS
IN_C = 128
OUT_C = 64
N_HEAD = 4
NEG_SLOPE = 0.2

NP = 8192
BP = 512
BS = 512
BD = 512
FA = N_HEAD * OUT_C
AUGC = FA + 8


def _prologue_body(z_ref, w_ref, al_ref, ar_ref, feat_ref, el_ref, ert_ref):
    z = z_ref[...]
    featf = jnp.dot(z, w_ref[...], preferred_element_type=jnp.float32)
    el_ref[...] = jnp.dot(featf, al_ref[...], preferred_element_type=jnp.float32)
    ert_ref[...] = lax.dot_general(
        ar_ref[...], featf, (((0,), (1,)), ((), ())),
        preferred_element_type=jnp.float32)
    fb = featf.astype(jnp.bfloat16)
    ones = jnp.ones((BP, 8), jnp.bfloat16)
    feat_ref[...] = jnp.concatenate([fb, ones], axis=1)


def _main_body(adj_ref, feat_ref, el_ref, ert_ref, bias_ref, out_ref,
               acc_ref, den_ref, *, ns):
    s = pl.program_id(1)

    @pl.when(s == 0)
    def _init():
        acc_ref[...] = jnp.zeros_like(acc_ref)
        den_ref[...] = jnp.zeros_like(den_ref)

    adj = adj_ref[...]
    row = lax.broadcasted_iota(jnp.int32, (BS, 1), 0) + s * BS
    edge = (adj == 1) & (row < N)
    maskneg = jnp.where(edge, 0.0, -1e30).astype(jnp.float32)

    feat = feat_ref[pl.ds(s * BS, BS), :]
    el = el_ref[...]
    ert = ert_ref[...]
    ones_col = feat[:, FA:FA + 8]
    for h in range(N_HEAD):
        e = el[:, h:h + 1] + ert[h:h + 1, :]
        e = jnp.maximum(e, NEG_SLOPE * e) + maskneg
        p = jnp.exp(e).astype(jnp.bfloat16)
        acc_ref[:, h * OUT_C:(h + 1) * OUT_C] += lax.dot_general(
            p, feat[:, h * OUT_C:(h + 1) * OUT_C],
            (((0,), (0,)), ((), ())), preferred_element_type=jnp.float32)
        den_ref[:, h * 8:(h + 1) * 8] += lax.dot_general(
            p, ones_col, (((0,), (0,)), ((), ())),
            preferred_element_type=jnp.float32)

    @pl.when(s == ns - 1)
    def _finalize():
        parts = []
        for h in range(N_HEAD):
            d = den_ref[:, h * 8:h * 8 + 1]
            parts.append(acc_ref[:, h * OUT_C:(h + 1) * OUT_C]
                         / jnp.maximum(d, 1e-16))
        r = jnp.concatenate(parts, axis=1) + bias_ref[...]
        out_ref[...] = jnp.where(r > 0, r, jnp.exp(r) - 1.0)


@jax.jit
def kernel(lncrna_x, disease_x, adj, W, attn_l, attn_r, bias):
    z = jnp.concatenate([lncrna_x, disease_x], axis=0)
    zp = jnp.pad(z, ((0, NP - N), (0, 0)))

    eye = jnp.eye(N_HEAD, 8, dtype=jnp.float32)
    a_l = (attn_l[:, :, None] * eye[:, None, :]).reshape(FA, 8)
    a_r = (attn_r[:, :, None] * eye[:, None, :]).reshape(FA, 8)

    feat, el, ert = pl.pallas_call(
        _prologue_body,
        grid=(NP // BP,),
        in_specs=[
            pl.BlockSpec((BP, IN_C), lambda i: (i, 0)),
            pl.BlockSpec((IN_C, FA), lambda i: (0, 0)),
            pl.BlockSpec((FA, 8), lambda i: (0, 0)),
            pl.BlockSpec((FA, 8), lambda i: (0, 0)),
        ],
        out_specs=[
            pl.BlockSpec((BP, AUGC), lambda i: (i, 0)),
            pl.BlockSpec((BP, 8), lambda i: (i, 0)),
            pl.BlockSpec((8, BP), lambda i: (0, i)),
        ],
        out_shape=[
            jax.ShapeDtypeStruct((NP, AUGC), jnp.bfloat16),
            jax.ShapeDtypeStruct((NP, 8), jnp.float32),
            jax.ShapeDtypeStruct((8, NP), jnp.float32),
        ],
    )(zp, W, a_l, a_r)

    nd, ns = NP // BD, NP // BS
    out = pl.pallas_call(
        functools.partial(_main_body, ns=ns),
        grid=(nd, ns),
        in_specs=[
            pl.BlockSpec((BS, BD), lambda d, s: (s, d)),
            pl.BlockSpec((NP, AUGC), lambda d, s: (0, 0)),
            pl.BlockSpec((BS, 8), lambda d, s: (s, 0)),
            pl.BlockSpec((8, BD), lambda d, s: (0, d)),
            pl.BlockSpec((1, FA), lambda d, s: (0, 0)),
        ],
        out_specs=pl.BlockSpec((BD, FA), lambda d, s: (d, 0)),
        out_shape=jax.ShapeDtypeStruct((NP, FA), jnp.float32),
        scratch_shapes=[
            pltpu.VMEM((BD, FA), jnp.float32),
            pltpu.VMEM((BD, 32), jnp.float32),
        ],
        compiler_params=pltpu.CompilerParams(
            dimension_semantics=("parallel", "arbitrary")),
    )(adj, feat, el, ert, bias.reshape(1, FA))
    return out[:N]
